# Initial kernel scaffold; baseline (speedup 1.0000x reference)
import jax
import jax.numpy as jnp
from jax.experimental import pallas as pl


def kernel(x, params):
    raise NotImplementedError("write your pallas kernel here")


# speedup vs baseline: 18.2021x; 18.2021x over previous
import jax
import jax.numpy as jnp
from jax.experimental import pallas as pl

_B, _N = 16, 4096
_NP1, _NS1, _R1 = 288, 8, 0.2
_NP2, _NS2, _R2 = 14, 4, 0.4
_NP1P = 384
_NP2P = 128
_SENT = 1.0e6
_BIG = 1.0e12


def _leaky(v):
    return jnp.where(v >= 0, v, 0.2 * v)


def _mlp(h, layers):
    for w, b in layers:
        h = _leaky(jnp.dot(h.astype(jnp.bfloat16), w.astype(jnp.bfloat16),
                           preferred_element_type=jnp.float32) + b)
    return h


def _first_index_of(value_eq, iota, n):
    return jnp.min(jnp.where(value_eq, iota, n), axis=1, keepdims=True)


def _fps_kernel(xt_ref, l1t_ref, l2t_ref):
    def run(x0, x1, x2, n, n_valid, npoint, npad):
        iota_n = jax.lax.broadcasted_iota(jnp.int32, (_B, n), 1)
        iota_p = jax.lax.broadcasted_iota(jnp.int32, (1, npad), 1)
        pad_p = (iota_p >= npoint).astype(jnp.float32) * _SENT

        def body(i, carry):
            distance, farth, a0, a1, a2 = carry
            oh = iota_n == farth
            c0 = jnp.sum(jnp.where(oh, x0, 0.0), axis=1, keepdims=True)
            c1 = jnp.sum(jnp.where(oh, x1, 0.0), axis=1, keepdims=True)
            c2 = jnp.sum(jnp.where(oh, x2, 0.0), axis=1, keepdims=True)
            sel = (iota_p == i).astype(jnp.float32)
            a0 = a0 + c0 * sel
            a1 = a1 + c1 * sel
            a2 = a2 + c2 * sel
            d = (x0 - c0) ** 2 + (x1 - c1) ** 2 + (x2 - c2) ** 2
            distance = jnp.minimum(distance, d)
            dmax = jnp.max(distance, axis=1, keepdims=True)
            farth = _first_index_of(distance == dmax, iota_n, n).astype(jnp.int32)
            return distance, farth, a0, a1, a2

        dist0 = jnp.where(iota_n < n_valid, jnp.float32(1e10), jnp.float32(-1.0))
        z = jnp.zeros((_B, npad), jnp.float32) + pad_p
        out = jax.lax.fori_loop(
            0, npoint, body,
            (dist0, jnp.zeros((_B, 1), jnp.int32), z, z, z),
        )
        return out[2], out[3], out[4]

    a0, a1, a2 = run(xt_ref[0], xt_ref[1], xt_ref[2], _N, _N, _NP1, _NP1P)
    l1t_ref[0], l1t_ref[1], l1t_ref[2] = a0, a1, a2
    b0, b1, b2 = run(a0, a1, a2, _NP1P, _NP1, _NP2, _NP2P)
    l2t_ref[0], l2t_ref[1], l2t_ref[2] = b0, b1, b2


def _ball_group_mlp(sq, src_feat, center_xyz, nsample, radius, layers):
    m, n = sq.shape
    iota = jax.lax.broadcasted_iota(jnp.int32, (m, n), 1)
    r2 = radius * radius
    sqw = sq
    nbr0 = None
    acc = None
    for j in range(nsample):
        dj = jnp.min(sqw, axis=1, keepdims=True)
        aj = _first_index_of(sqw == dj, iota, n)
        mask = iota == aj
        sqw = jnp.where(mask, jnp.inf, sqw)
        nbr = jnp.dot(mask.astype(jnp.float32), src_feat,
                      preferred_element_type=jnp.float32)
        if j == 0:
            nbr0 = nbr
        else:
            nbr = jnp.where(dj > r2, nbr0, nbr)
        g = jnp.concatenate([nbr[:, :3] - center_xyz, nbr[:, 3:]], axis=1)
        f = _mlp(g, layers)
        acc = f if j == 0 else jnp.maximum(acc, f)
    return acc


def _interp(sq, feat):
    m, n = sq.shape
    iota = jax.lax.broadcasted_iota(jnp.int32, (m, n), 1)
    sqw = sq
    ws, nbrs = [], []
    for _ in range(3):
        dj = jnp.min(sqw, axis=1, keepdims=True)
        aj = _first_index_of(sqw == dj, iota, n)
        mask = iota == aj
        sqw = jnp.where(mask, jnp.inf, sqw)
        ws.append(1.0 / (dj + 1e-8))
        nbrs.append(jnp.dot(mask.astype(jnp.float32), feat,
                            preferred_element_type=jnp.float32))
    wsum = (ws[0] + ws[2]) + ws[1]
    ts = [nb * (w / wsum) for w, nb in zip(ws, nbrs)]
    return (ts[0] + ts[2]) + ts[1]


def _zero_pad_rows(v, nvalid):
    ri = jax.lax.broadcasted_iota(jnp.int32, v.shape, 0)
    return jnp.where(ri < nvalid, v, 0.0)


def _main_kernel(x_ref, sq1_ref, sq2_ref, sq3_ref, sq4_ref,
                 l1_ref, l2_ref, *rest):
    param_refs, out_ref = rest[:-1], rest[-1]
    p = [r[...] for r in param_refs]
    sa1 = [(p[0], p[1]), (p[2], p[3]), (p[4], p[5])]
    sa2 = [(p[6], p[7]), (p[8], p[9]), (p[10], p[11])]
    fp3 = [(p[12], p[13]), (p[14], p[15])]
    fp4 = [(p[16], p[17]), (p[18], p[19]), (p[20], p[21])]
    d1 = (p[22], p[23])
    d2 = (p[24], p[25])

    xb = x_ref[0]
    l1xyz = l1_ref[0]
    l2xyz = l2_ref[0]

    feat1 = _ball_group_mlp(sq1_ref[0], xb, l1xyz, _NS1, _R1, sa1)
    feat1 = _zero_pad_rows(feat1, _NP1)

    l1cat = jnp.concatenate([l1xyz, feat1], axis=1)
    feat2 = _ball_group_mlp(sq2_ref[0], l1cat, l2xyz, _NS2, _R2, sa2)
    feat2 = _zero_pad_rows(feat2, _NP2)

    interp3 = _interp(sq3_ref[0], feat2)
    feat1b = _mlp(jnp.concatenate([feat1, interp3], axis=1), fp3)
    feat1b = _zero_pad_rows(feat1b, _NP1)

    interp4 = _interp(sq4_ref[0], feat1b)
    h = _mlp(jnp.concatenate([xb[:, 3:], interp4], axis=1), fp4)

    h = _leaky(jnp.dot(h.astype(jnp.bfloat16), d1[0].astype(jnp.bfloat16),
                       preferred_element_type=jnp.float32) + d1[1])
    out_ref[0] = jnp.dot(h.astype(jnp.bfloat16), d2[0].astype(jnp.bfloat16),
                         preferred_element_type=jnp.float32) + d2[1]


def _flatten_params(params):
    flat = []
    for w, b in params['sa1'] + params['sa2'] + params['fp3'] + params['fp4']:
        flat.append(w)
        flat.append(b.reshape(1, -1))
    for w, b in (params['d1'], params['d2']):
        flat.append(w)
        flat.append(b.reshape(1, -1))
    return flat


def _square_distance(a, b):
    return (jnp.sum(a * a, -1)[:, :, None] + jnp.sum(b * b, -1)[:, None, :]
            - 2.0 * jnp.einsum('bnc,bmc->bnm', a, b))


def _index_points(points, idx):
    b = jnp.arange(points.shape[0]).reshape((points.shape[0],) + (1,) * (idx.ndim - 1))
    return points[b, idx]


def _forward(x, params):
    xt = jnp.transpose(x[..., :3], (2, 0, 1))
    l1t, l2t = pl.pallas_call(
        _fps_kernel,
        out_shape=[
            jax.ShapeDtypeStruct((3, _B, _NP1P), jnp.float32),
            jax.ShapeDtypeStruct((3, _B, _NP2P), jnp.float32),
        ],
    )(xt)

    l0_xyz = x[..., :3]
    l1c = jnp.transpose(l1t, (1, 2, 0))[:, :_NP1]
    l2c = jnp.transpose(l2t, (1, 2, 0))[:, :_NP2]
    idx1 = jnp.argmax(jnp.all(l0_xyz[:, None, :, :] == l1c[:, :, None, :],
                              axis=-1), axis=-1).astype(jnp.int32)
    l1_xyz = _index_points(l0_xyz, idx1)
    idx2 = jnp.argmax(jnp.all(l1_xyz[:, None, :, :] == l2c[:, :, None, :],
                              axis=-1), axis=-1).astype(jnp.int32)
    l2_xyz = _index_points(l1_xyz, idx2)

    sq1 = _square_distance(l1_xyz, l0_xyz)
    sq2 = _square_distance(l2_xyz, l1_xyz)
    sq3 = _square_distance(l1_xyz, l2_xyz)
    sq4 = _square_distance(l0_xyz, l1_xyz)

    l1 = jnp.pad(l1_xyz, ((0, 0), (0, _NP1P - _NP1), (0, 0)), constant_values=_SENT)
    l2 = jnp.pad(l2_xyz, ((0, 0), (0, _NP2P - _NP2), (0, 0)), constant_values=_SENT)

    sq1p = jnp.pad(sq1, ((0, 0), (0, _NP1P - _NP1), (0, 0)), constant_values=_BIG)
    sq2p = jnp.pad(sq2, ((0, 0), (0, _NP2P - _NP2), (0, _NP1P - _NP1)),
                   constant_values=_BIG)
    sq3p = jnp.pad(sq3, ((0, 0), (0, _NP1P - _NP1), (0, _NP2P - _NP2)),
                   constant_values=_BIG)
    sq4p = jnp.pad(sq4, ((0, 0), (0, 0), (0, _NP1P - _NP1)), constant_values=_BIG)

    flat = _flatten_params(params)

    def bspec(shape):
        nd = len(shape)
        return pl.BlockSpec((1,) + shape[1:], lambda b, _nd=nd: (b,) + (0,) * (_nd - 1))

    def pspec(arr):
        nd = arr.ndim
        return pl.BlockSpec(arr.shape, lambda b, _nd=nd: (0,) * _nd)

    in_specs = [
        bspec((_B, _N, 9)),
        bspec((_B, _NP1P, _N)), bspec((_B, _NP2P, _NP1P)),
        bspec((_B, _NP1P, _NP2P)), bspec((_B, _N, _NP1P)),
        bspec((_B, _NP1P, 3)), bspec((_B, _NP2P, 3)),
    ] + [pspec(a) for a in flat]

    out = pl.pallas_call(
        _main_kernel,
        grid=(_B,),
        in_specs=in_specs,
        out_specs=bspec((_B, _N, 13)),
        out_shape=jax.ShapeDtypeStruct((_B, _N, 13), jnp.float32),
    )(x, sq1p, sq2p, sq3p, sq4p, l1, l2, *flat)
    return out


def kernel(x, params):
    return _forward(x, params)


# speedup vs baseline: 22.1237x; 1.2154x over previous
import jax
import jax.numpy as jnp
from jax.experimental import pallas as pl

_B, _N = 16, 4096
_NP1, _NS1, _R1 = 288, 8, 0.2
_NP2, _NS2, _R2 = 14, 4, 0.4
_NP1P = 384
_NP2P = 128
_SENT = 1.0e6


def _leaky(v):
    return jnp.where(v >= 0, v, 0.2 * v)


def _mlp(h, layers):
    for w, b in layers:
        h = _leaky(jnp.dot(h.astype(jnp.bfloat16), w.astype(jnp.bfloat16),
                           preferred_element_type=jnp.float32) + b)
    return h


def _first_index_of(value_eq, iota, n):
    return jnp.min(jnp.where(value_eq, iota, n), axis=1, keepdims=True)


def _fps_kernel(xt_ref, l1t_ref, l2t_ref):
    def run(x0, x1, x2, n, n_valid, npoint, npad):
        iota_n = jax.lax.broadcasted_iota(jnp.int32, (_B, n), 1)
        iota_p = jax.lax.broadcasted_iota(jnp.int32, (1, npad), 1)
        pad_p = (iota_p >= npoint).astype(jnp.float32) * _SENT

        def body(i, carry):
            distance, farth, a0, a1, a2 = carry
            oh = iota_n == farth
            c0 = jnp.sum(jnp.where(oh, x0, 0.0), axis=1, keepdims=True)
            c1 = jnp.sum(jnp.where(oh, x1, 0.0), axis=1, keepdims=True)
            c2 = jnp.sum(jnp.where(oh, x2, 0.0), axis=1, keepdims=True)
            sel = (iota_p == i).astype(jnp.float32)
            a0 = a0 + c0 * sel
            a1 = a1 + c1 * sel
            a2 = a2 + c2 * sel
            d = ((x0 - c0) ** 2 + (x2 - c2) ** 2) + (x1 - c1) ** 2
            distance = jnp.minimum(distance, d)
            dmax = jnp.max(distance, axis=1, keepdims=True)
            farth = _first_index_of(distance == dmax, iota_n, n).astype(jnp.int32)
            return distance, farth, a0, a1, a2

        dist0 = jnp.where(iota_n < n_valid, jnp.float32(1e10), jnp.float32(-1.0))
        z = jnp.zeros((_B, npad), jnp.float32) + pad_p
        out = jax.lax.fori_loop(
            0, npoint, body,
            (dist0, jnp.zeros((_B, 1), jnp.int32), z, z, z),
        )
        return out[2], out[3], out[4]

    a0, a1, a2 = run(xt_ref[0], xt_ref[1], xt_ref[2], _N, _N, _NP1, _NP1P)
    l1t_ref[0], l1t_ref[1], l1t_ref[2] = a0, a1, a2
    b0, b1, b2 = run(a0, a1, a2, _NP1P, _NP1, _NP2, _NP2P)
    l2t_ref[0], l2t_ref[1], l2t_ref[2] = b0, b1, b2


def _sq_dist(a, bt):
    sa = ((a[:, 0:1] * a[:, 0:1] + a[:, 2:3] * a[:, 2:3])
          + a[:, 1:2] * a[:, 1:2])
    sb = ((bt[0:1, :] * bt[0:1, :] + bt[2:3, :] * bt[2:3, :])
          + bt[1:2, :] * bt[1:2, :])
    cross = jnp.dot(a.astype(jnp.bfloat16), bt.astype(jnp.bfloat16),
                    preferred_element_type=jnp.float32)
    return (sa + sb) - 2.0 * cross


def _ball_group_mlp(sq, src_feat, center_xyz, nsample, radius, layers):
    m, n = sq.shape
    iota = jax.lax.broadcasted_iota(jnp.int32, (m, n), 1)
    r2 = radius * radius
    sqw = sq
    nbr0 = None
    acc = None
    for j in range(nsample):
        dj = jnp.min(sqw, axis=1, keepdims=True)
        aj = _first_index_of(sqw == dj, iota, n)
        mask = iota == aj
        sqw = jnp.where(mask, jnp.inf, sqw)
        nbr = jnp.dot(mask.astype(jnp.float32), src_feat,
                      preferred_element_type=jnp.float32)
        if j == 0:
            nbr0 = nbr
        else:
            nbr = jnp.where(dj > r2, nbr0, nbr)
        g = jnp.concatenate([nbr[:, :3] - center_xyz, nbr[:, 3:]], axis=1)
        f = _mlp(g, layers)
        acc = f if j == 0 else jnp.maximum(acc, f)
    return acc


def _interp(sq, feat):
    m, n = sq.shape
    iota = jax.lax.broadcasted_iota(jnp.int32, (m, n), 1)
    sqw = sq
    ws, nbrs = [], []
    for _ in range(3):
        dj = jnp.min(sqw, axis=1, keepdims=True)
        aj = _first_index_of(sqw == dj, iota, n)
        mask = iota == aj
        sqw = jnp.where(mask, jnp.inf, sqw)
        ws.append(1.0 / (dj + 1e-8))
        nbrs.append(jnp.dot(mask.astype(jnp.float32), feat,
                            preferred_element_type=jnp.float32))
    wsum = (ws[0] + ws[2]) + ws[1]
    ts = [nb * (w / wsum) for w, nb in zip(ws, nbrs)]
    return (ts[0] + ts[2]) + ts[1]


def _zero_pad_rows(v, nvalid):
    ri = jax.lax.broadcasted_iota(jnp.int32, v.shape, 0)
    return jnp.where(ri < nvalid, v, 0.0)


def _main_kernel(x_ref, xt_ref, l1_ref, l1t_ref, l2_ref, l2t_ref, *rest):
    param_refs, out_ref = rest[:-1], rest[-1]
    p = [r[...] for r in param_refs]
    sa1 = [(p[0], p[1]), (p[2], p[3]), (p[4], p[5])]
    sa2 = [(p[6], p[7]), (p[8], p[9]), (p[10], p[11])]
    fp3 = [(p[12], p[13]), (p[14], p[15])]
    fp4 = [(p[16], p[17]), (p[18], p[19]), (p[20], p[21])]
    d1 = (p[22], p[23])
    d2 = (p[24], p[25])

    xb = x_ref[0]
    xtb = xt_ref[0]
    l1xyz = l1_ref[0]
    l1t = l1t_ref[0]
    l2xyz = l2_ref[0]
    l2t = l2t_ref[0]

    sq1 = _sq_dist(l1xyz, xtb)
    feat1 = _ball_group_mlp(sq1, xb, l1xyz, _NS1, _R1, sa1)
    feat1 = _zero_pad_rows(feat1, _NP1)

    l1cat = jnp.concatenate([l1xyz, feat1], axis=1)
    sq2 = _sq_dist(l2xyz, l1t)
    feat2 = _ball_group_mlp(sq2, l1cat, l2xyz, _NS2, _R2, sa2)
    feat2 = _zero_pad_rows(feat2, _NP2)

    sq3 = _sq_dist(l1xyz, l2t)
    interp3 = _interp(sq3, feat2)
    feat1b = _mlp(jnp.concatenate([feat1, interp3], axis=1), fp3)
    feat1b = _zero_pad_rows(feat1b, _NP1)

    sq4 = _sq_dist(xb[:, :3], l1t)
    interp4 = _interp(sq4, feat1b)
    h = _mlp(jnp.concatenate([xb[:, 3:], interp4], axis=1), fp4)

    h = _leaky(jnp.dot(h.astype(jnp.bfloat16), d1[0].astype(jnp.bfloat16),
                       preferred_element_type=jnp.float32) + d1[1])
    out_ref[0] = jnp.dot(h.astype(jnp.bfloat16), d2[0].astype(jnp.bfloat16),
                         preferred_element_type=jnp.float32) + d2[1]


def _flatten_params(params):
    flat = []
    for w, b in params['sa1'] + params['sa2'] + params['fp3'] + params['fp4']:
        flat.append(w)
        flat.append(b.reshape(1, -1))
    for w, b in (params['d1'], params['d2']):
        flat.append(w)
        flat.append(b.reshape(1, -1))
    return flat


def _forward(x, params):
    xt = jnp.transpose(x[..., :3], (2, 0, 1))
    l1t, l2t = pl.pallas_call(
        _fps_kernel,
        out_shape=[
            jax.ShapeDtypeStruct((3, _B, _NP1P), jnp.float32),
            jax.ShapeDtypeStruct((3, _B, _NP2P), jnp.float32),
        ],
    )(xt)

    l1 = jnp.transpose(l1t, (1, 2, 0))
    l1t_b = jnp.transpose(l1t, (1, 0, 2))
    l2 = jnp.transpose(l2t, (1, 2, 0))
    l2t_b = jnp.transpose(l2t, (1, 0, 2))
    xt_b = jnp.transpose(xt, (1, 0, 2))

    flat = _flatten_params(params)

    def bspec(shape):
        nd = len(shape)
        return pl.BlockSpec((1,) + shape[1:], lambda b, _nd=nd: (b,) + (0,) * (_nd - 1))

    def pspec(arr):
        nd = arr.ndim
        return pl.BlockSpec(arr.shape, lambda b, _nd=nd: (0,) * _nd)

    in_specs = [
        bspec((_B, _N, 9)), bspec((_B, 3, _N)),
        bspec((_B, _NP1P, 3)), bspec((_B, 3, _NP1P)),
        bspec((_B, _NP2P, 3)), bspec((_B, 3, _NP2P)),
    ] + [pspec(a) for a in flat]

    out = pl.pallas_call(
        _main_kernel,
        grid=(_B,),
        in_specs=in_specs,
        out_specs=bspec((_B, _N, 13)),
        out_shape=jax.ShapeDtypeStruct((_B, _N, 13), jnp.float32),
    )(x, xt_b, l1, l1t_b, l2, l2t_b, *flat)
    return out


def kernel(x, params):
    return _forward(x, params)
